# baseline (speedup 1.0000x reference)
import functools

import jax
import jax.numpy as jnp
from jax import lax
from jax.experimental import pallas as pl
from jax.experimental.pallas import tpu as pltpu
from jax.experimental.pallas import tpu_sc as plsc

VOCAB = 1000000
D = 64
MAX_LEN = 256
B = 4096
S = 200

NC = 2
NS = 16
NW = NC * NS
ROWS = B * S
ROWS_W = ROWS // NW
SEQ_W = ROWS_W // S
LANES = 16

SPLIT = 128
REST = S - SPLIT


def _positional_encoding(max_len, d_model):
    position = jnp.arange(0, max_len, dtype=jnp.float32)[:, None]
    div_term = jnp.exp(
        jnp.arange(0, d_model, 2, dtype=jnp.float32)
        * -(jnp.log(jnp.float32(10000.0)) / d_model)
    )
    pe = jnp.zeros((max_len, d_model), dtype=jnp.float32)
    pe = pe.at[:, 0::2].set(jnp.sin(position * div_term))
    pe = pe.at[:, 1::2].set(jnp.cos(position * div_term))
    return pe


@functools.partial(
    pl.kernel,
    out_type=jax.ShapeDtypeStruct((ROWS, D), jnp.float32),
    mesh=plsc.VectorSubcoreMesh(core_axis_name="c", subcore_axis_name="s"),
    compiler_params=pltpu.CompilerParams(use_tc_tiling_on_sc=False),
    scratch_types=[
        pltpu.VMEM((SPLIT,), jnp.int32),
        pltpu.VMEM((REST,), jnp.int32),
        pltpu.VMEM((S, D), jnp.float32),
        pltpu.VMEM((S, D), jnp.float32),
        pltpu.SemaphoreType.DMA,
    ],
)
def _emb_lookup(x_hbm, tab_hbm, pe_hbm, out_hbm, idx_a, idx_b, rows_v, pe_v, sem):
    wid = lax.axis_index("s") * NC + lax.axis_index("c")
    pltpu.sync_copy(pe_hbm, pe_v)

    def chunk(j, carry):
        base = wid * ROWS_W + j * S
        pltpu.sync_copy(x_hbm.at[pl.ds(base, SPLIT)], idx_a)
        pltpu.sync_copy(x_hbm.at[pl.ds(base + SPLIT, REST)], idx_b)
        cp1 = pltpu.async_copy(tab_hbm.at[idx_a], rows_v.at[pl.ds(0, SPLIT)], sem)
        cp2 = pltpu.async_copy(tab_hbm.at[idx_b], rows_v.at[pl.ds(SPLIT, REST)], sem)
        cp1.wait()
        cp2.wait()

        def add_rows(r, carry2):
            r0 = r * 8
            for rr in range(8):
                for k in range(D // LANES):
                    sl = pl.ds(k * LANES, LANES)
                    plsc.addupdate(rows_v.at[r0 + rr, sl], pe_v[r0 + rr, sl])
            return carry2

        lax.fori_loop(0, S // 8, add_rows, 0)
        pltpu.sync_copy(rows_v, out_hbm.at[pl.ds(base, S)])
        return carry

    lax.fori_loop(0, SEQ_W, chunk, 0)


def kernel(x, embedding_weight):
    pe = _positional_encoding(MAX_LEN, D)[:S]
    out = _emb_lookup(x.reshape(ROWS), embedding_weight, pe)
    return out.reshape(B, S, D)


# speedup vs baseline: 1.1987x; 1.1987x over previous
import functools

import jax
import jax.numpy as jnp
from jax import lax
from jax.experimental import pallas as pl
from jax.experimental.pallas import tpu as pltpu
from jax.experimental.pallas import tpu_sc as plsc

VOCAB = 1000000
D = 64
MAX_LEN = 256
B = 4096
S = 200

NC = 2
NS = 16
NW = NC * NS
ROWS = B * S
ROWS_W = ROWS // NW
SEQ_W = ROWS_W // S
LANES = 16

SPLIT = 128
REST = S - SPLIT

NBUF = 4
ROUNDS = SEQ_W // NBUF


def _positional_encoding(max_len, d_model):
    position = jnp.arange(0, max_len, dtype=jnp.float32)[:, None]
    div_term = jnp.exp(
        jnp.arange(0, d_model, 2, dtype=jnp.float32)
        * -(jnp.log(jnp.float32(10000.0)) / d_model)
    )
    pe = jnp.zeros((max_len, d_model), dtype=jnp.float32)
    pe = pe.at[:, 0::2].set(jnp.sin(position * div_term))
    pe = pe.at[:, 1::2].set(jnp.cos(position * div_term))
    return pe


@functools.partial(
    pl.kernel,
    out_type=jax.ShapeDtypeStruct((ROWS, D), jnp.float32),
    mesh=plsc.VectorSubcoreMesh(core_axis_name="c", subcore_axis_name="s"),
    compiler_params=pltpu.CompilerParams(use_tc_tiling_on_sc=False),
    scratch_types=[
        pltpu.VMEM((SEQ_W, S), jnp.int32),
        pltpu.VMEM((S, D), jnp.float32),
        pltpu.VMEM((NBUF, S, D), jnp.float32),
    ]
    + [pltpu.SemaphoreType.DMA] * (2 * NBUF),
)
def _emb_lookup(x_hbm, tab_hbm, pe_hbm, out_hbm, idx_all, pe_v, rows_v, *sems):
    gsem = sems[:NBUF]
    wsem = sems[NBUF:]
    wid = lax.axis_index("s") * NC + lax.axis_index("c")

    pltpu.sync_copy(pe_hbm, pe_v)
    pltpu.sync_copy(x_hbm.at[wid], idx_all)

    def start_gather(j, b):
        pltpu.async_copy(
            tab_hbm.at[idx_all.at[j, pl.ds(0, SPLIT)]],
            rows_v.at[b, pl.ds(0, SPLIT)],
            gsem[b],
        )
        pltpu.async_copy(
            tab_hbm.at[idx_all.at[j, pl.ds(SPLIT, REST)]],
            rows_v.at[b, pl.ds(SPLIT, REST)],
            gsem[b],
        )

    def wait_gather(b):
        pltpu.make_async_copy(tab_hbm.at[pl.ds(0, S)], rows_v.at[b], gsem[b]).wait()

    def add_pe(b):
        def add_rows(r, carry):
            r0 = r * 8
            for rr in range(8):
                for k in range(D // LANES):
                    sl = pl.ds(k * LANES, LANES)
                    plsc.addupdate(rows_v.at[b, r0 + rr, sl], pe_v[r0 + rr, sl])
            return carry

        lax.fori_loop(0, S // 8, add_rows, 0)

    def do_seq(j, b, prefetch):
        wait_gather(b)
        add_pe(b)
        base = wid * ROWS_W + j * S
        cp = pltpu.async_copy(rows_v.at[b], out_hbm.at[pl.ds(base, S)], wsem[b])
        cp.wait()
        if prefetch:
            start_gather(j + NBUF, b)

    for b in range(NBUF):
        start_gather(b, b)

    def round_body(g, carry):
        for b in range(NBUF):
            do_seq(g * NBUF + b, b, prefetch=True)
        return carry

    lax.fori_loop(0, ROUNDS - 1, round_body, 0)
    for b in range(NBUF):
        do_seq((ROUNDS - 1) * NBUF + b, b, prefetch=False)


def kernel(x, embedding_weight):
    pe = _positional_encoding(MAX_LEN, D)[:S]
    out = _emb_lookup(x.reshape(NW, SEQ_W, S), embedding_weight, pe)
    return out.reshape(B, S, D)


# speedup vs baseline: 1.3065x; 1.0899x over previous
import functools

import jax
import jax.numpy as jnp
from jax import lax
from jax.experimental import pallas as pl
from jax.experimental.pallas import tpu as pltpu
from jax.experimental.pallas import tpu_sc as plsc

VOCAB = 1000000
D = 64
MAX_LEN = 256
B = 4096
S = 200

NC = 2
NS = 16
NW = NC * NS
LANES = 16

SPLIT = 128
OUT_TILES = S * 8 * 32
NBUF2 = 4
ROUNDS2 = S // NBUF2


def _positional_encoding(max_len, d_model):
    position = jnp.arange(0, max_len, dtype=jnp.float32)[:, None]
    div_term = jnp.exp(
        jnp.arange(0, d_model, 2, dtype=jnp.float32)
        * -(jnp.log(jnp.float32(10000.0)) / d_model)
    )
    pe = jnp.zeros((max_len, d_model), dtype=jnp.float32)
    pe = pe.at[:, 0::2].set(jnp.sin(position * div_term))
    pe = pe.at[:, 1::2].set(jnp.cos(position * div_term))
    return pe


@functools.partial(
    pl.kernel,
    out_type=jax.ShapeDtypeStruct((OUT_TILES, 8, SPLIT), jnp.float32),
    mesh=plsc.VectorSubcoreMesh(core_axis_name="c", subcore_axis_name="s"),
    compiler_params=pltpu.CompilerParams(
        use_tc_tiling_on_sc=False, needs_layout_passes=False
    ),
    scratch_types=[
        pltpu.VMEM((S, SPLIT), jnp.int32),
        pltpu.VMEM((S, D), jnp.float32),
        pltpu.VMEM((NBUF2, SPLIT, D), jnp.float32),
        pltpu.VMEM((NBUF2, D, 129), jnp.float32),
    ]
    + [pltpu.SemaphoreType.DMA] * (2 * NBUF2),
)
def _emb_lookup(xt_hbm, tab_hbm, pe_hbm, out_hbm, idx_v, pe_v, rows_v, blk_v, *sems):
    gsem = sems[:NBUF2]
    wsem = sems[NBUF2:]
    wid = lax.axis_index("s") * NC + lax.axis_index("c")

    pltpu.sync_copy(pe_hbm, pe_v)
    pltpu.sync_copy(xt_hbm.at[:, pl.ds(wid * SPLIT, SPLIT)], idx_v)

    scidx = [jnp.arange(LANES, dtype=jnp.int32) + k * LANES for k in range(D // LANES)]

    def start_gather(s, b):
        pltpu.async_copy(tab_hbm.at[idx_v.at[s]], rows_v.at[b], gsem[b])

    def wait_gather(b):
        pltpu.make_async_copy(tab_hbm.at[pl.ds(0, SPLIT)], rows_v.at[b], gsem[b]).wait()

    def scatter_pe(s, b):
        pe_k = [pe_v[s, pl.ds(k * LANES, LANES)] for k in range(D // LANES)]
        dst = blk_v.at[b]

        def body(jb, carry):
            jv = jnp.full((LANES,), jb, dtype=jnp.int32)
            for k in range(D // LANES):
                v = rows_v[b, jb, pl.ds(k * LANES, LANES)] + pe_k[k]
                plsc.store_scatter(dst, [scidx[k], jv], v)
            return carry

        lax.fori_loop(0, SPLIT, body, 0)

    def do_pos(s, b, prefetch):
        wait_gather(b)
        scatter_pe(s, b)
        cps = []
        for tr in range(8):
            cps.append(
                pltpu.async_copy(
                    blk_v.at[b, pl.ds(tr * 8, 8), pl.ds(0, SPLIT)],
                    out_hbm.at[(s * 8 + tr) * 32 + wid],
                    wsem[b],
                )
            )
        for cp in cps:
            cp.wait()
        if prefetch:
            start_gather(s + NBUF2, b)

    for b in range(NBUF2):
        start_gather(b, b)

    def round_body(g, carry):
        for b in range(NBUF2):
            do_pos(g * NBUF2 + b, b, prefetch=True)
        return carry

    lax.fori_loop(0, ROUNDS2 - 1, round_body, 0)
    for b in range(NBUF2):
        do_pos((ROUNDS2 - 1) * NBUF2 + b, b, prefetch=False)


def kernel(x, embedding_weight):
    pe = _positional_encoding(MAX_LEN, D)[:S]
    out = _emb_lookup(x.T, embedding_weight, pe)
    out = out.reshape(S, 8, 32, 8, 128).transpose(2, 4, 0, 1, 3)
    return out.reshape(B, S, D)
